# baseline (speedup 1.0000x reference)
import functools

import jax
import jax.numpy as jnp
from jax import lax
from jax.experimental import pallas as pl
from jax.experimental.pallas import tpu as pltpu
from jax.experimental.pallas import tpu_sc as plsc

H = 376
W = 376
HW = H * W
B = 4
BR = 64
PB = 24576
NB = (H + BR - 1) // BR
CODE = 8
QUAD = 4
LOC_WEIGHT = 2.0
FOCAL_ALPHA = 0.25

NW = 32
RANGE = 4480
FULL = (HW // 128) * 128
TAIL = HW - FULL

_L1P = (5.621959006663069e-07, 0.9999574870750698, -0.49920656854787626,
        0.3269731000139178, -0.22283625832784004, 0.13076503250360005,
        -0.05262485136716543, 0.010119082927575069)


def _log1p_poly(z):
    acc = jnp.float32(_L1P[7])
    for c in (_L1P[6], _L1P[5], _L1P[4], _L1P[3], _L1P[2], _L1P[1], _L1P[0]):
        acc = acc * z + jnp.float32(c)
    return acc


def _bce_sc_kernel(tf_hbm, sp_hbm, ql_hbm, out_hbm, tf_v, sp_v, ql_v,
                   tf_t, sp_t, ql_t, acc_v, sem):
    cid = lax.axis_index("c")
    sid = lax.axis_index("s")
    wid = sid * 2 + cid
    lo = wid * RANGE
    hi = jnp.minimum(lo + RANGE, FULL)
    base = pl.multiple_of(jnp.minimum(lo, FULL - RANGE), 128)

    cp_tf = pltpu.async_copy(tf_hbm.at[pl.ds(base, RANGE)], tf_v, sem)
    cp_sp = pltpu.async_copy(sp_hbm.at[:, pl.ds(base, RANGE)], sp_v, sem)
    cp_ql = [pltpu.async_copy(ql_hbm.at[b, :, pl.ds(base, RANGE)], ql_v.at[b], sem)
             for b in range(B)]
    cp_tf.wait()
    cp_sp.wait()
    for cp in cp_ql:
        cp.wait()

    def body(i, acc):
        o = i * 16
        pid = base + o + lax.iota(jnp.int32, 16)
        tfv = tf_v[pl.ds(o, 16)]
        mask = (tfv > 0.0) & (pid >= lo) & (pid < hi)
        for q in range(QUAD):
            x = sp_v[q, pl.ds(o, 16)]
            tsum = (ql_v[0, q, pl.ds(o, 16)] + ql_v[1, q, pl.ds(o, 16)]
                    + ql_v[2, q, pl.ds(o, 16)] + ql_v[3, q, pl.ds(o, 16)])
            z = jnp.exp(-jnp.abs(x))
            bce = jnp.maximum(x, 0.0) - x * tsum + _log1p_poly(z)
            acc = acc + jnp.where(mask, bce, jnp.float32(0.0))
        return acc

    acc = lax.fori_loop(0, RANGE // 16, body, jnp.zeros((16,), jnp.float32))
    acc_v[...] = acc

    @pl.when(wid == 0)
    def _tail():
        pltpu.sync_copy(tf_hbm.at[pl.ds(FULL, TAIL)], tf_t)
        pltpu.sync_copy(sp_hbm.at[:, pl.ds(FULL, TAIL)], sp_t)
        for b in range(B):
            pltpu.sync_copy(ql_hbm.at[b, :, pl.ds(FULL, TAIL)], ql_t.at[b])

        def tbody(i, acc):
            o = i * 16
            mask = tf_t[pl.ds(o, 16)] > 0.0
            for q in range(QUAD):
                x = sp_t[q, pl.ds(o, 16)]
                tsum = (ql_t[0, q, pl.ds(o, 16)] + ql_t[1, q, pl.ds(o, 16)]
                        + ql_t[2, q, pl.ds(o, 16)] + ql_t[3, q, pl.ds(o, 16)])
                z = jnp.exp(-jnp.abs(x))
                bce = jnp.maximum(x, 0.0) - x * tsum + _log1p_poly(z)
                acc = acc + jnp.where(mask, bce, jnp.float32(0.0))
            return acc

        acc_v[...] = lax.fori_loop(0, TAIL // 16, tbody, acc_v[...])

    pltpu.sync_copy(acc_v, out_hbm.at[wid])


def _tc_kernel(t_ref, cls_ref, tf_ref, bp_ref, hbl_ref, out_ref):
    s = pl.program_id(0)

    @pl.when(s == 0)
    def _init():
        for i in range(4):
            out_ref[i] = 0.0

    t = t_ref[...]
    rowok = (lax.broadcasted_iota(jnp.int32, (BR, W), 0) + s * BR) < H
    pos = (t > 0.0) & rowok
    m = pos | ((t == 0.0) & rowok)

    m_cnt = jnp.sum(m.astype(jnp.float32))
    n_pos = jnp.sum(pos.astype(jnp.float32))

    x = cls_ref[...]
    tb = t[None, :, :]
    z = jnp.exp(-jnp.abs(x))
    p = jnp.where(x >= 0.0, 1.0 / (1.0 + z), z / (1.0 + z))
    ce = jnp.maximum(x, 0.0) - x * tb + jnp.log(1.0 + z)
    p_t = p * tb + (1.0 - p) * (1.0 - tb)
    alpha_t = FOCAL_ALPHA * tb + (1.0 - FOCAL_ALPHA) * (1.0 - tb)
    om = 1.0 - p_t
    focal = alpha_t * om * om * ce
    s_focal = jnp.sum(jnp.where(m[None, :, :], focal, 0.0))

    tf = tf_ref[...]
    inb = (lax.iota(jnp.int32, PB) + s * PB) < HW
    mflat = ((tf > 0.0) & inb)[None, :]

    hbl = hbl_ref[...]
    hbls = hbl[0] + hbl[1] + hbl[2] + hbl[3]
    diff = bp_ref[...] - hbls
    ad = jnp.abs(diff)
    sl1 = jnp.where(ad < 1.0, 0.5 * diff * diff, ad - 0.5)
    s_sl1 = jnp.sum(jnp.where(mflat, sl1, 0.0))

    out_ref[0] += s_focal
    out_ref[1] += m_cnt
    out_ref[2] += n_pos
    out_ref[3] += s_sl1


def kernel(cls_preds, box_preds, spa_preds, heatmaps, hos_box_labels, quadrant_labels):
    t2 = heatmaps[0, 0]
    tflat = t2.reshape(HW)
    cls3 = cls_preds.reshape(B, H, W)
    bpT = box_preds.T
    hblT = jnp.transpose(hos_box_labels, (0, 1, 3, 2)).reshape(B, CODE, HW)
    spT = spa_preds.T
    qlT = jnp.transpose(quadrant_labels, (0, 1, 3, 2)).reshape(B, QUAD, HW)

    mesh = plsc.VectorSubcoreMesh(core_axis_name="c", subcore_axis_name="s")
    sc_bce = functools.partial(
        pl.kernel,
        out_type=jax.ShapeDtypeStruct((NW, 16), jnp.float32),
        mesh=mesh,
        scratch_types=[
            pltpu.VMEM((RANGE,), jnp.float32),
            pltpu.VMEM((QUAD, RANGE), jnp.float32),
            pltpu.VMEM((B, QUAD, RANGE), jnp.float32),
            pltpu.VMEM((TAIL,), jnp.float32),
            pltpu.VMEM((QUAD, TAIL), jnp.float32),
            pltpu.VMEM((B, QUAD, TAIL), jnp.float32),
            pltpu.VMEM((16,), jnp.float32),
            pltpu.SemaphoreType.DMA,
        ],
    )(_bce_sc_kernel)
    bce_parts = sc_bce(tflat, spT, qlT)

    tc = pl.pallas_call(
        _tc_kernel,
        grid=(NB,),
        in_specs=[
            pl.BlockSpec((BR, W), lambda s: (s, 0)),
            pl.BlockSpec((B, BR, W), lambda s: (0, s, 0)),
            pl.BlockSpec((PB,), lambda s: (s,)),
            pl.BlockSpec((CODE, PB), lambda s: (0, s)),
            pl.BlockSpec((B, CODE, PB), lambda s: (0, 0, s)),
        ],
        out_specs=pl.BlockSpec(memory_space=pltpu.SMEM),
        out_shape=jax.ShapeDtypeStruct((4,), jnp.float32),
    )(t2, cls3, tflat, bpT, hblT)

    n_pos = tc[2]
    cls_loss = tc[0] / jnp.maximum(tc[1], 1.0)
    reg_loss = tc[3] / jnp.maximum(n_pos, 1.0) * LOC_WEIGHT
    spa_loss = jnp.sum(bce_parts) / jnp.maximum(n_pos * QUAD, 1.0)
    return cls_loss + reg_loss + spa_loss


# speedup vs baseline: 1.8783x; 1.8783x over previous
import jax
import jax.numpy as jnp
from jax import lax
from jax.experimental import pallas as pl
from jax.experimental.pallas import tpu as pltpu

H = 376
W = 376
HW = H * W
B = 4
BR = 96
PB = 35840
NB = (H + BR - 1) // BR
CODE = 8
QUAD = 4
LOC_WEIGHT = 2.0
FOCAL_ALPHA = 0.25


def _loss_kernel(t_ref, cls_ref, tf_ref, bp_ref, hbl_ref, sp_ref, ql_ref, out_ref):
    s = pl.program_id(0)

    @pl.when(s == 0)
    def _init():
        for i in range(6):
            out_ref[i] = 0.0

    t = t_ref[...]
    rowok = (lax.broadcasted_iota(jnp.int32, (BR, W), 0) + s * BR) < H
    pos = (t > 0.0) & rowok
    m = pos | ((t == 0.0) & rowok)

    m_cnt = jnp.sum(m.astype(jnp.float32))
    n_pos = jnp.sum(pos.astype(jnp.float32))

    x = cls_ref[...]
    tb = t[None, :, :]
    z = jnp.exp(-jnp.abs(x))
    p = jnp.where(x >= 0.0, 1.0 / (1.0 + z), z / (1.0 + z))
    ce = jnp.maximum(x, 0.0) - x * tb + jnp.log(1.0 + z)
    p_t = p * tb + (1.0 - p) * (1.0 - tb)
    alpha_t = FOCAL_ALPHA * tb + (1.0 - FOCAL_ALPHA) * (1.0 - tb)
    om = 1.0 - p_t
    focal = alpha_t * om * om * ce
    s_focal = jnp.sum(jnp.where(m[None, :, :], focal, 0.0))

    tf = tf_ref[...]
    inb = (lax.iota(jnp.int32, PB) + s * PB) < HW
    mflat = ((tf > 0.0) & inb)[None, :]

    hbl = hbl_ref[...]
    hbls = hbl[0] + hbl[1] + hbl[2] + hbl[3]
    diff = bp_ref[...] - hbls
    ad = jnp.abs(diff)
    sl1 = jnp.where(ad < 1.0, 0.5 * diff * diff, ad - 0.5)
    s_sl1 = jnp.sum(jnp.where(mflat, sl1, 0.0))

    ql = ql_ref[...]
    qls = ql[0] + ql[1] + ql[2] + ql[3]
    spv = sp_ref[...]
    bce = (jnp.maximum(spv, 0.0) - spv * qls
           + jnp.log(1.0 + jnp.exp(-jnp.abs(spv))))
    s_bce = jnp.sum(jnp.where(mflat, bce, 0.0))

    out_ref[0] += s_focal
    out_ref[1] += m_cnt
    out_ref[2] += n_pos
    out_ref[3] += s_sl1
    out_ref[4] += s_bce

    @pl.when(s == NB - 1)
    def _finish():
        cls_loss = out_ref[0] / jnp.maximum(out_ref[1], 1.0)
        reg_loss = out_ref[3] / jnp.maximum(out_ref[2], 1.0) * LOC_WEIGHT
        spa_loss = out_ref[4] / jnp.maximum(out_ref[2] * QUAD, 1.0)
        out_ref[5] = cls_loss + reg_loss + spa_loss


def kernel(cls_preds, box_preds, spa_preds, heatmaps, hos_box_labels, quadrant_labels):
    t2 = heatmaps[0, 0]
    tflat = t2.reshape(HW)
    cls3 = cls_preds.reshape(B, H, W)
    bpT = box_preds.T
    hblT = jnp.transpose(hos_box_labels, (0, 1, 3, 2)).reshape(B, CODE, HW)
    spT = spa_preds.T
    qlT = jnp.transpose(quadrant_labels, (0, 1, 3, 2)).reshape(B, QUAD, HW)

    out = pl.pallas_call(
        _loss_kernel,
        grid=(NB,),
        in_specs=[
            pl.BlockSpec((BR, W), lambda s: (s, 0)),
            pl.BlockSpec((B, BR, W), lambda s: (0, s, 0)),
            pl.BlockSpec((PB,), lambda s: (s,)),
            pl.BlockSpec((CODE, PB), lambda s: (0, s)),
            pl.BlockSpec((B, CODE, PB), lambda s: (0, 0, s)),
            pl.BlockSpec((QUAD, PB), lambda s: (0, s)),
            pl.BlockSpec((B, QUAD, PB), lambda s: (0, 0, s)),
        ],
        out_specs=pl.BlockSpec(memory_space=pltpu.SMEM),
        out_shape=jax.ShapeDtypeStruct((6,), jnp.float32),
    )(t2, cls3, tflat, bpT, hblT, spT, qlT)
    return out[5]
